# baseline (speedup 1.0000x reference)
import functools

import jax
import jax.numpy as jnp
from jax import lax
from jax.experimental import pallas as pl
from jax.experimental.pallas import tpu as pltpu
from jax.experimental.pallas import tpu_sc as plsc

VOCAB = 1000000
B = 16384
L = 20
DIM = 64
H1 = 128
H2 = 64

NC = 2
NS = 16
NW = NC * NS

BPC = 4
RPC = BPC * L
BW = B // NW
CW = BW // BPC
KV = DIM // 16

_mesh = plsc.VectorSubcoreMesh(
    core_axis_name="c", subcore_axis_name="s", num_cores=NC, num_subcores=NS
)


NBUF = 4


def _pool_body(table, idx_hbm, out_hbm, idx_v, rows, pooled, sems):
  wid = lax.axis_index("s") * NC + lax.axis_index("c")

  def accum(c, b):
    for j in range(BPC):
      for k in range(KV):
        acc = rows[b][j * L, pl.ds(k * 16, 16)]
        for el in range(1, L):
          acc = acc + rows[b][j * L + el, pl.ds(k * 16, 16)]
        pooled[c * BPC + j, pl.ds(k * 16, 16)] = acc

  pltpu.sync_copy(idx_hbm.at[pl.ds(wid * CW, CW)], idx_v)
  for b in range(NBUF):
    pltpu.make_async_copy(table.at[idx_v.at[b]], rows[b], sems[b]).start()

  def body(i, carry):
    for b in range(NBUF):
      c = NBUF * i + b
      pltpu.make_async_copy(table.at[idx_v.at[c]], rows[b], sems[b]).wait()
      accum(c, b)

      @pl.when(i < CW // NBUF - 1)
      def _():
        pltpu.make_async_copy(
            table.at[idx_v.at[c + NBUF]], rows[b], sems[b]).start()

    return carry

  lax.fori_loop(0, CW // NBUF, body, 0)
  pltpu.sync_copy(pooled, out_hbm.at[pl.ds(wid * BW, BW)])


_pool = functools.partial(
    pl.kernel,
    out_type=jax.ShapeDtypeStruct((B, DIM), jnp.float32),
    mesh=_mesh,
    scratch_types=[
        pltpu.VMEM((CW, RPC), jnp.int32),
        [pltpu.VMEM((RPC, DIM), jnp.float32) for _ in range(NBUF)],
        pltpu.VMEM((BW, DIM), jnp.float32),
        [pltpu.SemaphoreType.DMA for _ in range(NBUF)],
    ],
    compiler_params=pltpu.CompilerParams(use_tc_tiling_on_sc=False),
)(_pool_body)


TCH = 8192
TGRID = -(-VOCAB // TCH)


def _relayout_body(x_ref, o_ref):
  sub = 512
  for k in range(0, TCH // 2, sub):
    xx = jnp.concatenate(
        [x_ref[:, k:k + sub], x_ref[:, TCH // 2 + k:TCH // 2 + k + sub]],
        axis=0)
    o_ref[pl.ds(k, sub), :] = jnp.transpose(xx)


_relayout = pl.pallas_call(
    _relayout_body,
    grid=(TGRID,),
    in_specs=[pl.BlockSpec((DIM, TCH), lambda i: (0, i))],
    out_specs=pl.BlockSpec((TCH // 2, 128), lambda i: (i, 0)),
    out_shape=jax.ShapeDtypeStruct((TGRID * TCH // 2, 128), jnp.float32),
)


def _rowmajor(table):
  return _relayout(table.T).reshape(TGRID * TCH, DIM)


def _view_idx(v):
  r = v % TCH
  return v - r + 2 * (r % (TCH // 2)) + (r // (TCH // 2))


BM = 2048


def _mlp_body(x, w1, b1, w2, b2, o_ref):
  h = jnp.maximum(
      jnp.dot(x[...], w1[...], preferred_element_type=jnp.float32) + b1[...],
      0.0)
  o_ref[...] = jnp.maximum(
      jnp.dot(h, w2[...], preferred_element_type=jnp.float32) + b2[...],
      0.0)


def _full(shape):
  return pl.BlockSpec(shape, lambda i: (0, 0))


_mlp = pl.pallas_call(
    _mlp_body,
    grid=(B // BM,),
    in_specs=[
        pl.BlockSpec((BM, DIM), lambda i: (i, 0)),
        _full((DIM, H1)),
        _full((1, H1)),
        _full((H1, H2)),
        _full((1, H2)),
    ],
    out_specs=pl.BlockSpec((BM, H2), lambda i: (i, 0)),
    out_shape=jax.ShapeDtypeStruct((B, H2), jnp.float32),
)


def kernel(user_table, item_table, Wu1, bu1, Wu2, bu2, Wi1, bi1, Wi2, bi2,
           user_idx, item_idx):
  u2 = _view_idx(user_idx).reshape(B * L // RPC, RPC)
  i2 = _view_idx(item_idx).reshape(B * L // RPC, RPC)
  pu = _pool(_rowmajor(user_table), u2)
  pi = _pool(_rowmajor(item_table), i2)
  q = _mlp(pu, Wu1, bu1.reshape(1, H1), Wu2, bu2.reshape(1, H2))
  c = _mlp(pi, Wi1, bi1.reshape(1, H1), Wi2, bi2.reshape(1, H2))
  return (q, c)


# speedup vs baseline: 1.0583x; 1.0583x over previous
import functools

import jax
import jax.numpy as jnp
from jax import lax
from jax.experimental import pallas as pl
from jax.experimental.pallas import tpu as pltpu
from jax.experimental.pallas import tpu_sc as plsc

VOCAB = 1000000
B = 16384
L = 20
DIM = 64
H1 = 128
H2 = 64

NC = 2
NS = 16
NW = NC * NS

BPC = 4
RPC = BPC * L
BW = B // NW
CW = BW // BPC
KV = DIM // 16

_mesh = plsc.VectorSubcoreMesh(
    core_axis_name="c", subcore_axis_name="s", num_cores=NC, num_subcores=NS
)


NBUF = 4


def _pool_body(table, idx_hbm, out_hbm, idx_v, rows, pooled, sems):
  wid = lax.axis_index("s") * NC + lax.axis_index("c")

  def accum(c, b):
    for j in range(BPC):
      for k in range(KV):
        vals = [rows[b][j * L + el, pl.ds(k * 16, 16)] for el in range(L)]
        while len(vals) > 1:
          nxt = [vals[t] + vals[t + 1] for t in range(0, len(vals) - 1, 2)]
          if len(vals) % 2:
            nxt.append(vals[-1])
          vals = nxt
        pooled[c * BPC + j, pl.ds(k * 16, 16)] = vals[0]

  pltpu.sync_copy(idx_hbm.at[pl.ds(wid * CW, CW)], idx_v)
  for b in range(NBUF):
    pltpu.make_async_copy(table.at[idx_v.at[b]], rows[b], sems[b]).start()

  def body(i, carry):
    for b in range(NBUF):
      c = NBUF * i + b
      pltpu.make_async_copy(table.at[idx_v.at[c]], rows[b], sems[b]).wait()
      accum(c, b)
      pltpu.make_async_copy(
          table.at[idx_v.at[c + NBUF]], rows[b], sems[b]).start()
    return carry

  lax.fori_loop(0, CW // NBUF - 1, body, 0)
  for b in range(NBUF):
    c = CW - NBUF + b
    pltpu.make_async_copy(table.at[idx_v.at[c]], rows[b], sems[b]).wait()
    accum(c, b)
  pltpu.sync_copy(pooled, out_hbm.at[pl.ds(wid * BW, BW)])


_pool = functools.partial(
    pl.kernel,
    out_type=jax.ShapeDtypeStruct((B, DIM), jnp.float32),
    mesh=_mesh,
    scratch_types=[
        pltpu.VMEM((CW, RPC), jnp.int32),
        [pltpu.VMEM((RPC, DIM), jnp.float32) for _ in range(NBUF)],
        pltpu.VMEM((BW, DIM), jnp.float32),
        [pltpu.SemaphoreType.DMA for _ in range(NBUF)],
    ],
    compiler_params=pltpu.CompilerParams(use_tc_tiling_on_sc=False),
)(_pool_body)


TCH = 8192
TGRID = -(-VOCAB // TCH)


def _relayout_body(x_ref, o_ref):
  sub = 512
  for k in range(0, TCH // 2, sub):
    xx = jnp.concatenate(
        [x_ref[:, k:k + sub], x_ref[:, TCH // 2 + k:TCH // 2 + k + sub]],
        axis=0)
    o_ref[pl.ds(k, sub), :] = jnp.transpose(xx)


_relayout = pl.pallas_call(
    _relayout_body,
    grid=(TGRID,),
    in_specs=[pl.BlockSpec((DIM, TCH), lambda i: (0, i))],
    out_specs=pl.BlockSpec((TCH // 2, 128), lambda i: (i, 0)),
    out_shape=jax.ShapeDtypeStruct((TGRID * TCH // 2, 128), jnp.float32),
)


def _rowmajor(table):
  return _relayout(table.T).reshape(TGRID * TCH, DIM)


def _view_idx(v):
  r = v % TCH
  return v - r + 2 * (r % (TCH // 2)) + (r // (TCH // 2))


BM = 2048


def _mlp_body(x, w1, b1, w2, b2, o_ref):
  h = jnp.maximum(
      jnp.dot(x[...], w1[...], preferred_element_type=jnp.float32) + b1[...],
      0.0)
  o_ref[...] = jnp.maximum(
      jnp.dot(h, w2[...], preferred_element_type=jnp.float32) + b2[...],
      0.0)


def _full(shape):
  return pl.BlockSpec(shape, lambda i: (0, 0))


_mlp = pl.pallas_call(
    _mlp_body,
    grid=(B // BM,),
    in_specs=[
        pl.BlockSpec((BM, DIM), lambda i: (i, 0)),
        _full((DIM, H1)),
        _full((1, H1)),
        _full((H1, H2)),
        _full((1, H2)),
    ],
    out_specs=pl.BlockSpec((BM, H2), lambda i: (i, 0)),
    out_shape=jax.ShapeDtypeStruct((B, H2), jnp.float32),
)


def kernel(user_table, item_table, Wu1, bu1, Wu2, bu2, Wi1, bi1, Wi2, bi2,
           user_idx, item_idx):
  u2 = _view_idx(user_idx).reshape(B * L // RPC, RPC)
  i2 = _view_idx(item_idx).reshape(B * L // RPC, RPC)
  pu = _pool(_rowmajor(user_table), u2)
  pi = _pool(_rowmajor(item_table), i2)
  q = _mlp(pu, Wu1, bu1.reshape(1, H1), Wu2, bu2.reshape(1, H2))
  c = _mlp(pi, Wi1, bi1.reshape(1, H1), Wi2, bi2.reshape(1, H2))
  return (q, c)


# speedup vs baseline: 1.1491x; 1.0858x over previous
import functools

import jax
import jax.numpy as jnp
from jax import lax
from jax.experimental import pallas as pl
from jax.experimental.pallas import tpu as pltpu
from jax.experimental.pallas import tpu_sc as plsc

VOCAB = 1000000
B = 16384
L = 20
DIM = 64
H1 = 128
H2 = 64

NC = 2
NS = 16
NW = NC * NS

BPC = 4
RPC = BPC * L
BW = B // NW
CW = BW // BPC
KV = DIM // 16

_mesh = plsc.VectorSubcoreMesh(
    core_axis_name="c", subcore_axis_name="s", num_cores=NC, num_subcores=NS
)


NBUF = 4


def _pool_body(table, idx_hbm, out_hbm, idx_v, rows, pooled, sems):
  wid = lax.axis_index("s") * NC + lax.axis_index("c")

  def accum(c, b):
    for j in range(BPC):
      for k in range(KV):
        vals = [rows[b][j * L + el, pl.ds(k * 16, 16)] for el in range(L)]
        while len(vals) > 1:
          nxt = [vals[t] + vals[t + 1] for t in range(0, len(vals) - 1, 2)]
          if len(vals) % 2:
            nxt.append(vals[-1])
          vals = nxt
        pooled[c * BPC + j, pl.ds(k * 16, 16)] = vals[0]

  pltpu.sync_copy(idx_hbm.at[pl.ds(wid * CW, CW)], idx_v)
  for b in range(NBUF):
    pltpu.make_async_copy(table.at[idx_v.at[b]], rows[b], sems[b]).start()

  def body(i, carry):
    for b in range(NBUF):
      c = NBUF * i + b
      pltpu.make_async_copy(table.at[idx_v.at[c]], rows[b], sems[b]).wait()
      accum(c, b)
      pltpu.make_async_copy(
          table.at[idx_v.at[c + NBUF]], rows[b], sems[b]).start()
    return carry

  lax.fori_loop(0, CW // NBUF - 1, body, 0)
  for b in range(NBUF):
    c = CW - NBUF + b
    pltpu.make_async_copy(table.at[idx_v.at[c]], rows[b], sems[b]).wait()
    accum(c, b)
  pltpu.sync_copy(pooled, out_hbm.at[pl.ds(wid * BW, BW)])


_pool = functools.partial(
    pl.kernel,
    out_type=jax.ShapeDtypeStruct((B, DIM), jnp.float32),
    mesh=_mesh,
    scratch_types=[
        pltpu.VMEM((CW, RPC), jnp.int32),
        [pltpu.VMEM((RPC, DIM), jnp.float32) for _ in range(NBUF)],
        pltpu.VMEM((BW, DIM), jnp.float32),
        [pltpu.SemaphoreType.DMA for _ in range(NBUF)],
    ],
    compiler_params=pltpu.CompilerParams(use_tc_tiling_on_sc=False),
)(_pool_body)


TCH = 16384
TGRID = -(-VOCAB // TCH)


def _relayout_body(x_ref, o_ref):
  sub = 512
  for k in range(0, TCH // 2, sub):
    xx = jnp.concatenate(
        [x_ref[:, k:k + sub], x_ref[:, TCH // 2 + k:TCH // 2 + k + sub]],
        axis=0)
    o_ref[pl.ds(k, sub), :] = jnp.transpose(xx)


_relayout = pl.pallas_call(
    _relayout_body,
    grid=(TGRID,),
    in_specs=[pl.BlockSpec((DIM, TCH), lambda i: (0, i))],
    out_specs=pl.BlockSpec((TCH // 2, 128), lambda i: (i, 0)),
    out_shape=jax.ShapeDtypeStruct((TGRID * TCH // 2, 128), jnp.float32),
)


def _rowmajor(table):
  return _relayout(table.T).reshape(TGRID * TCH, DIM)


def _view_idx(v):
  r = v % TCH
  return v - r + 2 * (r % (TCH // 2)) + (r // (TCH // 2))


BM = 2048


def _mlp_body(x, w1, b1, w2, b2, o_ref):
  h = jnp.maximum(
      jnp.dot(x[...], w1[...], preferred_element_type=jnp.float32) + b1[...],
      0.0)
  o_ref[...] = jnp.maximum(
      jnp.dot(h, w2[...], preferred_element_type=jnp.float32) + b2[...],
      0.0)


def _full(shape):
  return pl.BlockSpec(shape, lambda i: (0, 0))


_mlp = pl.pallas_call(
    _mlp_body,
    grid=(B // BM,),
    in_specs=[
        pl.BlockSpec((BM, DIM), lambda i: (i, 0)),
        _full((DIM, H1)),
        _full((1, H1)),
        _full((H1, H2)),
        _full((1, H2)),
    ],
    out_specs=pl.BlockSpec((BM, H2), lambda i: (i, 0)),
    out_shape=jax.ShapeDtypeStruct((B, H2), jnp.float32),
)


def kernel(user_table, item_table, Wu1, bu1, Wu2, bu2, Wi1, bi1, Wi2, bi2,
           user_idx, item_idx):
  u2 = _view_idx(user_idx).reshape(B * L // RPC, RPC)
  i2 = _view_idx(item_idx).reshape(B * L // RPC, RPC)
  pu = _pool(_rowmajor(user_table), u2)
  pi = _pool(_rowmajor(item_table), i2)
  q = _mlp(pu, Wu1, bu1.reshape(1, H1), Wu2, bu2.reshape(1, H2))
  c = _mlp(pi, Wi1, bi1.reshape(1, H1), Wi2, bi2.reshape(1, H2))
  return (q, c)


# speedup vs baseline: 1.1715x; 1.0195x over previous
import functools

import jax
import jax.numpy as jnp
from jax import lax
from jax.experimental import pallas as pl
from jax.experimental.pallas import tpu as pltpu
from jax.experimental.pallas import tpu_sc as plsc

VOCAB = 1000000
B = 16384
L = 20
DIM = 64
H1 = 128
H2 = 64

NC = 2
NS = 16
NW = NC * NS

BPC = 4
RPC = BPC * L
BW = B // NW
CW = BW // BPC
KV = DIM // 16

_mesh = plsc.VectorSubcoreMesh(
    core_axis_name="c", subcore_axis_name="s", num_cores=NC, num_subcores=NS
)


NBUF = 4


def _pool_body(table, idx_hbm, out_hbm, idx_v, rows, pooled, sems):
  wid = lax.axis_index("s") * NC + lax.axis_index("c")

  def accum(c, b):
    for j in range(BPC):
      for k in range(KV):
        vals = [rows[b][j * L + el, pl.ds(k * 16, 16)] for el in range(L)]
        while len(vals) > 1:
          nxt = [vals[t] + vals[t + 1] for t in range(0, len(vals) - 1, 2)]
          if len(vals) % 2:
            nxt.append(vals[-1])
          vals = nxt
        pooled[c * BPC + j, pl.ds(k * 16, 16)] = vals[0]

  pltpu.sync_copy(idx_hbm.at[pl.ds(wid * CW, CW)], idx_v)
  for b in range(NBUF):
    pltpu.make_async_copy(table.at[idx_v.at[b]], rows[b], sems[b]).start()

  def body(i, carry):
    for b in range(NBUF):
      c = NBUF * i + b
      pltpu.make_async_copy(table.at[idx_v.at[c]], rows[b], sems[b]).wait()
      accum(c, b)
      pltpu.make_async_copy(
          table.at[idx_v.at[c + NBUF]], rows[b], sems[b]).start()
    return carry

  lax.fori_loop(0, CW // NBUF - 1, body, 0)
  for b in range(NBUF):
    c = CW - NBUF + b
    pltpu.make_async_copy(table.at[idx_v.at[c]], rows[b], sems[b]).wait()
    accum(c, b)
  pltpu.sync_copy(pooled, out_hbm.at[pl.ds(wid * BW, BW)])


_pool = functools.partial(
    pl.kernel,
    out_type=jax.ShapeDtypeStruct((B, DIM), jnp.float32),
    mesh=_mesh,
    scratch_types=[
        pltpu.VMEM((CW, RPC), jnp.int32),
        [pltpu.VMEM((RPC, DIM), jnp.float32) for _ in range(NBUF)],
        pltpu.VMEM((BW, DIM), jnp.float32),
        [pltpu.SemaphoreType.DMA for _ in range(NBUF)],
    ],
    compiler_params=pltpu.CompilerParams(use_tc_tiling_on_sc=False),
)(_pool_body)


TCH = 32768
TGRID = -(-VOCAB // TCH)


def _relayout_body(x_ref, o_ref):
  sub = 512
  for k in range(0, TCH // 2, sub):
    xx = jnp.concatenate(
        [x_ref[:, k:k + sub], x_ref[:, TCH // 2 + k:TCH // 2 + k + sub]],
        axis=0)
    o_ref[pl.ds(k, sub), :] = jnp.transpose(xx)


_relayout = pl.pallas_call(
    _relayout_body,
    grid=(TGRID,),
    in_specs=[pl.BlockSpec((DIM, TCH), lambda i: (0, i))],
    out_specs=pl.BlockSpec((TCH // 2, 128), lambda i: (i, 0)),
    out_shape=jax.ShapeDtypeStruct((TGRID * TCH // 2, 128), jnp.float32),
)


def _rowmajor(table):
  return _relayout(table.T).reshape(TGRID * TCH, DIM)


def _view_idx(v):
  r = v % TCH
  return v - r + 2 * (r % (TCH // 2)) + (r // (TCH // 2))


BM = 2048


def _mlp_body(x, w1, b1, w2, b2, o_ref):
  h = jnp.maximum(
      jnp.dot(x[...], w1[...], preferred_element_type=jnp.float32) + b1[...],
      0.0)
  o_ref[...] = jnp.maximum(
      jnp.dot(h, w2[...], preferred_element_type=jnp.float32) + b2[...],
      0.0)


def _full(shape):
  return pl.BlockSpec(shape, lambda i: (0, 0))


_mlp = pl.pallas_call(
    _mlp_body,
    grid=(B // BM,),
    in_specs=[
        pl.BlockSpec((BM, DIM), lambda i: (i, 0)),
        _full((DIM, H1)),
        _full((1, H1)),
        _full((H1, H2)),
        _full((1, H2)),
    ],
    out_specs=pl.BlockSpec((BM, H2), lambda i: (i, 0)),
    out_shape=jax.ShapeDtypeStruct((B, H2), jnp.float32),
)


def kernel(user_table, item_table, Wu1, bu1, Wu2, bu2, Wi1, bi1, Wi2, bi2,
           user_idx, item_idx):
  u2 = _view_idx(user_idx).reshape(B * L // RPC, RPC)
  i2 = _view_idx(item_idx).reshape(B * L // RPC, RPC)
  pu = _pool(_rowmajor(user_table), u2)
  pi = _pool(_rowmajor(item_table), i2)
  q = _mlp(pu, Wu1, bu1.reshape(1, H1), Wu2, bu2.reshape(1, H2))
  c = _mlp(pi, Wi1, bi1.reshape(1, H1), Wi2, bi2.reshape(1, H2))
  return (q, c)
